# Initial kernel scaffold; baseline (speedup 1.0000x reference)
import jax
import jax.numpy as jnp
from jax.experimental import pallas as pl


def kernel(x, edge_index, batch, W1, b1, W2, b2, g1, be1, W3, b3, W4, b4, g2, be2, W5, b5):
    raise NotImplementedError("write your pallas kernel here")


# speedup vs baseline: 4.8488x; 4.8488x over previous
import functools

import jax
import jax.numpy as jnp
from jax import lax
from jax.experimental import pallas as pl
from jax.experimental.pallas import tpu as pltpu
from jax.experimental.pallas import tpu_sc as plsc

N = 10000
E = 320000
D = 128
G = 64

NC = 2
NS = 16
NW = NC * NS
EPW = E // NW
EK = 80
NITER = EPW // EK
RPS = N // NS
ZR = 125
WR = 632


def _agg_body(src_hbm, dst_hbm, h_hbm, out_hbm, acc, src_v, dst_v, rows_v,
              zbuf, sem):
  cid = lax.axis_index("c")
  sid = lax.axis_index("s")
  wid = cid * NS + sid

  def _zrow(i, _):
    for j in range(D // 16):
      zbuf[i, pl.ds(j * 16, 16)] = jnp.zeros((16,), jnp.float32)
    return _
  lax.fori_loop(0, ZR, _zrow, None)
  for r in range(RPS // ZR):
    pltpu.sync_copy(zbuf, acc.at[pl.ds(sid * RPS + r * ZR, ZR)])
  plsc.subcore_barrier()

  def _edge_chunk(it, _):
    base = wid * EPW + it * EK
    pltpu.sync_copy(src_hbm.at[pl.ds(base, EK)], src_v)
    pltpu.sync_copy(dst_hbm.at[pl.ds(base, EK)], dst_v)
    pltpu.async_copy(h_hbm.at[src_v], rows_v, sem).wait()
    pltpu.sync_copy(rows_v, acc.at[dst_v], add=True)
    return _
  lax.fori_loop(0, NITER, _edge_chunk, None)
  plsc.subcore_barrier()

  s8 = (sid * RPS) // 8 * 8
  pltpu.sync_copy(acc.at[pl.ds(s8, WR)], out_hbm.at[cid, pl.ds(s8, WR)])


_edge_agg = functools.partial(
    pl.kernel,
    out_type=jax.ShapeDtypeStruct((NC, N, D), jnp.float32),
    mesh=plsc.VectorSubcoreMesh(core_axis_name="c", subcore_axis_name="s"),
    scratch_types=[
        pltpu.VMEM_SHARED((N, D), jnp.float32),
        pltpu.VMEM((EK,), jnp.int32),
        pltpu.VMEM((EK,), jnp.int32),
        pltpu.VMEM((EK, D), jnp.float32),
        pltpu.VMEM((ZR, D), jnp.float32),
        pltpu.SemaphoreType.DMA,
    ],
)(_agg_body)


def _mlp_bn_body(x_ref, a_ref, wa_ref, ba_ref, wb_ref, bb_ref, g_ref, be_ref,
                 o_ref):
  z = x_ref[...] + a_ref[0] + a_ref[1]
  z = jnp.dot(z, wa_ref[...], preferred_element_type=jnp.float32) + ba_ref[...]
  z = jnp.maximum(z, 0.0)
  z = jnp.dot(z, wb_ref[...], preferred_element_type=jnp.float32) + bb_ref[...]
  mu = jnp.mean(z, axis=0, keepdims=True)
  zc = z - mu
  var = jnp.mean(zc * zc, axis=0, keepdims=True)
  z = zc * lax.rsqrt(var + 1e-5) * g_ref[...] + be_ref[...]
  o_ref[...] = jnp.maximum(z, 0.0)


def _mlp_bn(x, a, wa, ba, wb, bb, g, be):
  return pl.pallas_call(
      _mlp_bn_body,
      out_shape=jax.ShapeDtypeStruct((N, D), jnp.float32),
  )(x, a, wa, ba, wb, bb, g, be)


def _mlp_bn_pool_body(x_ref, a_ref, wa_ref, ba_ref, wb_ref, bb_ref, g_ref,
                      be_ref, batch_ref, wc_ref, bc_ref, o_ref):
  z = x_ref[...] + a_ref[0] + a_ref[1]
  z = jnp.dot(z, wa_ref[...], preferred_element_type=jnp.float32) + ba_ref[...]
  z = jnp.maximum(z, 0.0)
  z = jnp.dot(z, wb_ref[...], preferred_element_type=jnp.float32) + bb_ref[...]
  mu = jnp.mean(z, axis=0, keepdims=True)
  zc = z - mu
  var = jnp.mean(zc * zc, axis=0, keepdims=True)
  z = zc * lax.rsqrt(var + 1e-5) * g_ref[...] + be_ref[...]
  z = jnp.maximum(z, 0.0)

  seg = (batch_ref[...] == lax.broadcasted_iota(jnp.int32, (N, G), 1))
  seg = seg.astype(jnp.float32)
  sums = lax.dot_general(seg, z, (((0,), (0,)), ((), ())),
                         preferred_element_type=jnp.float32)
  counts = lax.dot_general(seg, jnp.ones((N, 1), jnp.float32),
                           (((0,), (0,)), ((), ())),
                           preferred_element_type=jnp.float32)
  pooled = sums / jnp.maximum(counts, 1.0)
  logits = jnp.dot(pooled, wc_ref[...],
                   preferred_element_type=jnp.float32) + bc_ref[...]
  s = logits - jnp.max(logits, axis=1, keepdims=True)
  o_ref[...] = s - jnp.log(jnp.sum(jnp.exp(s), axis=1, keepdims=True))


def _mlp_bn_pool(x, a, wa, ba, wb, bb, g, be, batch2d, wc, bc):
  return pl.pallas_call(
      _mlp_bn_pool_body,
      out_shape=jax.ShapeDtypeStruct((G, wc.shape[1]), jnp.float32),
  )(x, a, wa, ba, wb, bb, g, be, batch2d, wc, bc)


@jax.jit
def kernel(x, edge_index, batch, W1, b1, W2, b2, g1, be1, W3, b3, W4, b4, g2,
           be2, W5, b5):
  src = edge_index[0]
  dst = edge_index[1]
  batch2d = batch.reshape(N, 1)
  r = lambda v: v.reshape(1, -1)

  a1 = _edge_agg(src, dst, x)
  h1 = _mlp_bn(x, a1, W1, r(b1), W2, r(b2), r(g1), r(be1))
  a2 = _edge_agg(src, dst, h1)
  return _mlp_bn_pool(h1, a2, W3, r(b3), W4, r(b4), r(g2), r(be2), batch2d,
                      W5, r(b5))


# speedup vs baseline: 5.6229x; 1.1596x over previous
import functools

import jax
import jax.numpy as jnp
from jax import lax
from jax.experimental import pallas as pl
from jax.experimental.pallas import tpu as pltpu
from jax.experimental.pallas import tpu_sc as plsc

N = 10000
E = 320000
D = 128
G = 64

NC = 2
NS = 16
NW = NC * NS
EK = 128
NITER = -(-E // (NW * EK))
EPW = NITER * EK
EPAD = NW * EPW - E
NPAD = N + 8
RPS = N // NS
ZR = 25
WR = 632


def _agg_body(src_hbm, dst_hbm, h_hbm, out_hbm, acc, s0, s1, d0, d1, buf0,
              buf1, zbuf, semi0, semi1, semg0, semg1):
  cid = lax.axis_index("c")
  sid = lax.axis_index("s")
  wid = cid * NS + sid
  s = [s0, s1]
  d = [d0, d1]
  buf = [buf0, buf1]
  semi = [semi0, semi1]
  semg = [semg0, semg1]

  def _issue_idx(c, p):
    base = wid * EPW + c * EK
    pltpu.async_copy(src_hbm.at[pl.ds(base, EK)], s[p], semi[p])
    pltpu.async_copy(dst_hbm.at[pl.ds(base, EK)], d[p], semi[p])

  def _wait_idx(p):
    pltpu.make_async_copy(src_hbm.at[pl.ds(0, EK)], s[p], semi[p]).wait()
    pltpu.make_async_copy(src_hbm.at[pl.ds(0, EK)], d[p], semi[p]).wait()

  def _issue_gather(p):
    pltpu.async_copy(h_hbm.at[s[p]], buf[p], semg[p])

  def _wait_gather(p):
    pltpu.make_async_copy(h_hbm.at[s[p]], buf[p], semg[p]).wait()

  def _step(c, p, has_next, has_fetch):
    if has_next:
      _wait_idx(1 - p)
      _issue_gather(1 - p)
    _wait_gather(p)
    pltpu.sync_copy(buf[p], acc.at[d[p]], add=True)
    if has_fetch:
      _issue_idx(c + 2, p)

  _issue_idx(0, 0)
  _issue_idx(1, 1)

  def _zrow(i, c):
    for j in range(D // 16):
      zbuf[i, pl.ds(j * 16, 16)] = jnp.zeros((16,), jnp.float32)
    return c
  lax.fori_loop(0, ZR, _zrow, None)
  for r in range(RPS // ZR):
    pltpu.sync_copy(zbuf, acc.at[pl.ds(sid * RPS + r * ZR, ZR)])

  _wait_idx(0)
  _issue_gather(0)
  plsc.subcore_barrier()

  def _pair(j, c):
    _step(2 * j, 0, True, True)
    _step(2 * j + 1, 1, True, True)
    return c
  lax.fori_loop(0, (NITER - 5) // 2, _pair, None)
  for c in range(NITER - 5, NITER):
    _step(c, c & 1, c + 1 < NITER, c + 2 < NITER)
  plsc.subcore_barrier()

  s8 = (sid * RPS) // 8 * 8
  pltpu.sync_copy(acc.at[pl.ds(s8, WR)], out_hbm.at[cid, pl.ds(s8, WR)])


_edge_agg = functools.partial(
    pl.kernel,
    out_type=jax.ShapeDtypeStruct((NC, N, D), jnp.float32),
    mesh=plsc.VectorSubcoreMesh(core_axis_name="c", subcore_axis_name="s"),
    scratch_types=[
        pltpu.VMEM_SHARED((NPAD, D), jnp.float32),
        pltpu.VMEM((EK,), jnp.int32),
        pltpu.VMEM((EK,), jnp.int32),
        pltpu.VMEM((EK,), jnp.int32),
        pltpu.VMEM((EK,), jnp.int32),
        pltpu.VMEM((EK, D), jnp.float32),
        pltpu.VMEM((EK, D), jnp.float32),
        pltpu.VMEM((ZR, D), jnp.float32),
        pltpu.SemaphoreType.DMA,
        pltpu.SemaphoreType.DMA,
        pltpu.SemaphoreType.DMA,
        pltpu.SemaphoreType.DMA,
    ],
)(_agg_body)


def _mlp_bn_body(x_ref, a_ref, wa_ref, ba_ref, wb_ref, bb_ref, g_ref, be_ref,
                 o_ref):
  z = x_ref[...] + a_ref[0] + a_ref[1]
  z = jnp.dot(z, wa_ref[...], preferred_element_type=jnp.float32) + ba_ref[...]
  z = jnp.maximum(z, 0.0)
  z = jnp.dot(z, wb_ref[...], preferred_element_type=jnp.float32) + bb_ref[...]
  mu = jnp.mean(z, axis=0, keepdims=True)
  zc = z - mu
  var = jnp.mean(zc * zc, axis=0, keepdims=True)
  z = zc * lax.rsqrt(var + 1e-5) * g_ref[...] + be_ref[...]
  o_ref[...] = jnp.maximum(z, 0.0)


def _mlp_bn(x, a, wa, ba, wb, bb, g, be):
  return pl.pallas_call(
      _mlp_bn_body,
      out_shape=jax.ShapeDtypeStruct((N, D), jnp.float32),
  )(x, a, wa, ba, wb, bb, g, be)


def _mlp_bn_pool_body(x_ref, a_ref, wa_ref, ba_ref, wb_ref, bb_ref, g_ref,
                      be_ref, batch_ref, wc_ref, bc_ref, o_ref):
  z = x_ref[...] + a_ref[0] + a_ref[1]
  z = jnp.dot(z, wa_ref[...], preferred_element_type=jnp.float32) + ba_ref[...]
  z = jnp.maximum(z, 0.0)
  z = jnp.dot(z, wb_ref[...], preferred_element_type=jnp.float32) + bb_ref[...]
  mu = jnp.mean(z, axis=0, keepdims=True)
  zc = z - mu
  var = jnp.mean(zc * zc, axis=0, keepdims=True)
  z = zc * lax.rsqrt(var + 1e-5) * g_ref[...] + be_ref[...]
  z = jnp.maximum(z, 0.0)

  seg = (batch_ref[...] == lax.broadcasted_iota(jnp.int32, (N, G), 1))
  seg = seg.astype(jnp.float32)
  sums = lax.dot_general(seg, z, (((0,), (0,)), ((), ())),
                         preferred_element_type=jnp.float32)
  counts = lax.dot_general(seg, jnp.ones((N, 1), jnp.float32),
                           (((0,), (0,)), ((), ())),
                           preferred_element_type=jnp.float32)
  pooled = sums / jnp.maximum(counts, 1.0)
  logits = jnp.dot(pooled, wc_ref[...],
                   preferred_element_type=jnp.float32) + bc_ref[...]
  s = logits - jnp.max(logits, axis=1, keepdims=True)
  o_ref[...] = s - jnp.log(jnp.sum(jnp.exp(s), axis=1, keepdims=True))


def _mlp_bn_pool(x, a, wa, ba, wb, bb, g, be, batch2d, wc, bc):
  return pl.pallas_call(
      _mlp_bn_pool_body,
      out_shape=jax.ShapeDtypeStruct((G, wc.shape[1]), jnp.float32),
  )(x, a, wa, ba, wb, bb, g, be, batch2d, wc, bc)


@jax.jit
def kernel(x, edge_index, batch, W1, b1, W2, b2, g1, be1, W3, b3, W4, b4, g2,
           be2, W5, b5):
  src = jnp.concatenate([edge_index[0], jnp.zeros((EPAD,), jnp.int32)])
  dst = jnp.concatenate([edge_index[1], jnp.full((EPAD,), N, jnp.int32)])
  batch2d = batch.reshape(N, 1)
  r = lambda v: v.reshape(1, -1)

  a1 = _edge_agg(src, dst, x)
  h1 = _mlp_bn(x, a1, W1, r(b1), W2, r(b2), r(g1), r(be1))
  a2 = _edge_agg(src, dst, h1)
  return _mlp_bn_pool(h1, a2, W3, r(b3), W4, r(b4), r(g2), r(be2), batch2d,
                      W5, r(b5))
